# baseline (speedup 1.0000x reference)
import functools

import jax
import jax.numpy as jnp
from jax import lax
from jax.experimental import pallas as pl
from jax.experimental.pallas import tpu as pltpu
from jax.experimental.pallas import tpu_sc as plsc

_NC, _NS, _L = 2, 16, 16


@functools.lru_cache(maxsize=None)
def _make_sc(B, S, D, nt, dummy_outs=0):
    NW = _NC * _NS
    wpw = S // NW
    CH = 4
    CE = CH * D
    npc = wpw // CH
    npt = B * npc
    NB = 8
    K = 4
    cshift = (D // _L).bit_length() - 1
    cmask = (D // _L) - 1
    pshift = npc.bit_length() - 1
    pmask = npc - 1

    mesh = plsc.VectorSubcoreMesh(core_axis_name="c", subcore_axis_name="s")
    out_t = jax.ShapeDtypeStruct((B * S, D), jnp.float32)

    @functools.partial(
        pl.kernel,
        mesh=mesh,
        out_type=[out_t] * (nt + dummy_outs),
        scratch_types=(
            [pltpu.VMEM((CH, D), jnp.float32) for _ in range(NB)]
            + [pltpu.VMEM((wpw, D), jnp.float32)]
            + [pltpu.SemaphoreType.DMA for _ in range(2 * NB)]
        ),
    )
    def k(*args):
        ins = args[:nt]
        p_hbm = args[nt]
        outs = args[nt + 1:2 * nt + 1]
        scratch = args[2 * nt + 1 + dummy_outs:]
        xb = scratch[:NB]
        pall = scratch[NB]
        in_s = scratch[NB + 1:NB + 1 + NB]
        out_s = scratch[NB + 1 + NB:]

        wid = lax.axis_index("s") * _NC + lax.axis_index("c")
        prow0 = wid * wpw

        def rbase(ci):
            b = lax.shift_right_logical(ci, pshift)
            pc = lax.bitwise_and(ci, pmask)
            return b * S + prow0 + pc * CH

        def prow(ci):
            return lax.bitwise_and(ci, pmask) * CH

        def compute(q, ci):
            xq = xb[q]
            pr = prow(ci)

            @plsc.parallel_loop(0, CE // _L, unroll=8)
            def _(i):
                r = lax.shift_right_logical(i, cshift)
                c = pl.multiple_of(
                    lax.shift_left(lax.bitwise_and(i, cmask), 4), _L)
                plsc.addupdate(xq.at[r, pl.ds(c, _L)],
                               pall[pr + r, pl.ds(c, _L)])

        pltpu.sync_copy(p_hbm.at[pl.ds(prow0, wpw)], pall)

        for t in range(nt):
            x_hbm, o_hbm = ins[t], outs[t]

            def issue_in(ci, q):
                pltpu.async_copy(x_hbm.at[pl.ds(rbase(ci), CH)],
                                 xb[q], in_s[q])

            def wait_in(q):
                pltpu.make_async_copy(
                    x_hbm.at[pl.ds(0, CH)], xb[q], in_s[q]).wait()

            def issue_out(ci, q):
                pltpu.async_copy(xb[q], o_hbm.at[pl.ds(rbase(ci), CH)],
                                 out_s[q])

            def wait_out(q):
                pltpu.make_async_copy(
                    xb[q], o_hbm.at[pl.ds(0, CH)], out_s[q]).wait()

            for q in range(K):
                issue_in(q, q)

            n_it = npt // NB

            @pl.loop(0, n_it)
            def _(j):
                for q in range(NB):
                    ci = j * NB + q
                    wait_in(q)
                    compute(q, ci)
                    issue_out(ci, q)
                    nq = (q + K) % NB
                    if q < NB - K:
                        @pl.when(j > 0)
                        def _():
                            wait_out(nq)
                        issue_in(ci + K, nq)
                    else:
                        @pl.when(j < n_it - 1)
                        def _():
                            wait_out(nq)
                            issue_in(ci + K, nq)

            for q in range(NB):
                wait_out(q)

    return k


def _tc_body2(x0_ref, x1_ref, pos_ref, o0, o1):
    p = pos_ref[...]
    o0[0] = x0_ref[0] + p
    o1[0] = x1_ref[0] + p


def _tc_body3(x0_ref, x1_ref, x2_ref, pos_ref, o0, o1, o2):
    p = pos_ref[...]
    o0[0] = x0_ref[0] + p
    o1[0] = x1_ref[0] + p
    o2[0] = x2_ref[0] + p


@functools.lru_cache(maxsize=None)
def _make_tc(B, S, D, nt, BS=512):
    x_spec = pl.BlockSpec((1, BS, D), lambda s, b: (b, s, 0))
    pos_spec = pl.BlockSpec((BS, D), lambda s, b: (s, 0))
    out_shape = jax.ShapeDtypeStruct((B, S, D), jnp.float32)
    body = {2: _tc_body2, 3: _tc_body3}[nt]
    return pl.pallas_call(
        body,
        grid=(S // BS, B),
        in_specs=[x_spec] * nt + [pos_spec],
        out_specs=[x_spec] * nt,
        out_shape=[out_shape] * nt,
    )


def kernel(rtg, state, action, pos_table):
    B, S, D = rtg.shape
    pos = pos_table[:S]
    (o2,) = _make_sc(B, S, D, 1)(action.reshape(B * S, D), pos)
    o0, o1 = _make_tc(B, S, D, 2, BS=1024)(rtg, state, pos)
    return (o0, o1, o2.reshape(B, S, D))
